# baseline (speedup 1.0000x reference)
import functools

import jax
import jax.numpy as jnp
from jax import lax
from jax.experimental import pallas as pl
from jax.experimental.pallas import tpu as pltpu
from jax.experimental.pallas import tpu_sc as plsc

_V = 100000
_S = 200
_B = 1024
_LANES = 16
_SPAD = 208
_NCH = _SPAD // _LANES
_NVALID = _S - (_NCH - 1) * _LANES
_NW = 32
_PPW = _B // _NW
_IDS_PER_W = _PPW * _SPAD
_NGRP = 4
_PPG = _PPW // _NGRP
_IDS_PER_G = _PPG * _SPAD


def _body(ids_hbm, w_hbm, bias_hbm, out_hbm,
          ids_v, vals_v, scratch_v, outbuf_v, acc_v, bias_v,
          wshared, sem0, sem1, sem2, sem3):
    sems = (sem0, sem1, sem2, sem3)
    sid = lax.axis_index("s")
    wid = sid * 2 + lax.axis_index("c")
    base = wid * _IDS_PER_W

    @pl.when(sid == 0)
    def _stage_w():
        pltpu.sync_copy(w_hbm, wshared)

    pltpu.sync_copy(ids_hbm.at[pl.ds(base, _IDS_PER_W)], ids_v)
    pltpu.sync_copy(bias_hbm, bias_v)
    plsc.subcore_barrier()
    gathers = [
        pltpu.async_copy(
            wshared.at[ids_v.at[pl.ds(g * _IDS_PER_G, _IDS_PER_G)]],
            vals_v.at[pl.ds(g * _IDS_PER_G, _IDS_PER_G)], sems[g])
        for g in range(_NGRP)
    ]
    lane = lax.iota(jnp.int32, _LANES)
    valid_last = lane < _NVALID
    bvec = bias_v[...]

    def phrase(p, carry):
        pbase = p * _SPAD
        ids = []
        for k in range(_NCH):
            ids16 = ids_v[pl.ds(pbase + k * _LANES, _LANES)]
            ids.append(ids16)
            tags16 = lane + (pbase + k * _LANES)
            if k == _NCH - 1:
                plsc.store_scatter(scratch_v, [ids16], tags16,
                                   mask=valid_last)
            else:
                plsc.store_scatter(scratch_v, [ids16], tags16)
        acc = jnp.zeros((_LANES,), jnp.float32)
        for k in range(_NCH):
            tags16 = lane + (pbase + k * _LANES)
            r16 = plsc.load_gather(scratch_v, [ids[k]])
            v16 = vals_v[pl.ds(pbase + k * _LANES, _LANES)]
            m16 = r16 == tags16
            if k == _NCH - 1:
                m16 = m16 & valid_last
            acc = acc + jnp.where(m16, v16, 0.0)
        acc_v[pl.ds(p * _LANES, _LANES)] = acc
        return carry

    for g in range(_NGRP):
        gathers[g].wait()
        lax.fori_loop(g * _PPG, (g + 1) * _PPG, phrase, 0)
    for p in range(_PPW):
        tot = jnp.sum(acc_v[pl.ds(p * _LANES, _LANES)])
        out16 = jnp.full((_LANES,), tot, jnp.float32) + bvec
        plsc.store_scatter(outbuf_v, [jnp.full((_LANES,), p, jnp.int32)],
                           out16, mask=lane == 0)
    pltpu.sync_copy(outbuf_v, out_hbm.at[pl.ds(wid * _PPW, _PPW)])


_mnb_sc = functools.partial(
    pl.kernel,
    out_type=jax.ShapeDtypeStruct((_B,), jnp.float32),
    mesh=plsc.VectorSubcoreMesh(core_axis_name="c", subcore_axis_name="s"),
    compiler_params=pltpu.CompilerParams(needs_layout_passes=False),
    scratch_types=[
        pltpu.VMEM((_IDS_PER_W,), jnp.int32),
        pltpu.VMEM((_IDS_PER_W,), jnp.float32),
        pltpu.VMEM((_V,), jnp.int32),
        pltpu.VMEM((_PPW,), jnp.float32),
        pltpu.VMEM((_PPW * _LANES,), jnp.float32),
        pltpu.VMEM((_LANES,), jnp.float32),
        pltpu.VMEM_SHARED((_V,), jnp.float32),
        pltpu.SemaphoreType.DMA,
        pltpu.SemaphoreType.DMA,
        pltpu.SemaphoreType.DMA,
        pltpu.SemaphoreType.DMA,
    ],
)(_body)


@jax.jit
def kernel(text, W, b):
    ids = text.astype(jnp.int32).T
    pad = jnp.zeros((_B, _SPAD - _S), jnp.int32)
    ids_flat = jnp.concatenate([ids, pad], axis=1).reshape(_B * _SPAD)
    bias16 = jnp.broadcast_to(b.astype(jnp.float32), (_LANES,))
    out = _mnb_sc(ids_flat, W.reshape(_V), bias16)
    return out.reshape(_B, 1)


# speedup vs baseline: 1.0090x; 1.0090x over previous
import functools

import jax
import jax.numpy as jnp
from jax import lax
from jax.experimental import pallas as pl
from jax.experimental.pallas import tpu as pltpu
from jax.experimental.pallas import tpu_sc as plsc

_V = 100000
_S = 200
_B = 1024
_LANES = 16
_SPAD = 208
_NCH = _SPAD // _LANES
_NVALID = _S - (_NCH - 1) * _LANES
_NW = 32
_PPW = _B // _NW
_IDS_PER_W = _PPW * _SPAD
_NGRP = 4
_PPG = _PPW // _NGRP
_IDS_PER_G = _PPG * _SPAD


def _body(ids_hbm, w_hbm, bias_hbm, out_hbm,
          ids_v, vals_v, scratch_v, outbuf_v, bias_v,
          wshared, sem0, sem1, sem2, sem3):
    sems = (sem0, sem1, sem2, sem3)
    sid = lax.axis_index("s")
    wid = sid * 2 + lax.axis_index("c")
    base = wid * _IDS_PER_W

    @pl.when(sid == 0)
    def _stage_w():
        pltpu.sync_copy(w_hbm, wshared)

    pltpu.sync_copy(ids_hbm.at[pl.ds(base, _IDS_PER_W)], ids_v)
    pltpu.sync_copy(bias_hbm, bias_v)
    plsc.subcore_barrier()
    gathers = [
        pltpu.async_copy(
            wshared.at[ids_v.at[pl.ds(g * _IDS_PER_G, _IDS_PER_G)]],
            vals_v.at[pl.ds(g * _IDS_PER_G, _IDS_PER_G)], sems[g])
        for g in range(_NGRP)
    ]
    lane = lax.iota(jnp.int32, _LANES)
    valid_last = lane < _NVALID
    bvec = bias_v[...]

    def phrase(p, carry):
        pbase = p * _SPAD
        ids = []
        for k in range(_NCH):
            ids16 = ids_v[pl.ds(pbase + k * _LANES, _LANES)]
            ids.append(ids16)
            tags16 = lane + (pbase + k * _LANES)
            if k == _NCH - 1:
                plsc.store_scatter(scratch_v, [ids16], tags16,
                                   mask=valid_last)
            else:
                plsc.store_scatter(scratch_v, [ids16], tags16)
        acc = jnp.zeros((_LANES,), jnp.float32)
        for k in range(_NCH):
            tags16 = lane + (pbase + k * _LANES)
            r16 = plsc.load_gather(scratch_v, [ids[k]])
            v16 = vals_v[pl.ds(pbase + k * _LANES, _LANES)]
            m16 = r16 == tags16
            if k == _NCH - 1:
                m16 = m16 & valid_last
            acc = acc + jnp.where(m16, v16, 0.0)
        tot = jnp.sum(acc)
        out16 = jnp.full((_LANES,), tot, jnp.float32) + bvec
        plsc.store_scatter(outbuf_v, [jnp.full((_LANES,), p, jnp.int32)],
                           out16, mask=lane == 0)
        return carry

    for g in range(_NGRP):
        gathers[g].wait()
        lax.fori_loop(g * _PPG, (g + 1) * _PPG, phrase, 0)
    pltpu.sync_copy(outbuf_v, out_hbm.at[pl.ds(wid * _PPW, _PPW)])


_mnb_sc = functools.partial(
    pl.kernel,
    out_type=jax.ShapeDtypeStruct((_B,), jnp.float32),
    mesh=plsc.VectorSubcoreMesh(core_axis_name="c", subcore_axis_name="s"),
    compiler_params=pltpu.CompilerParams(
        needs_layout_passes=False,
        disable_bounds_checks=True,
        skip_device_barrier=True,
    ),
    scratch_types=[
        pltpu.VMEM((_IDS_PER_W,), jnp.int32),
        pltpu.VMEM((_IDS_PER_W,), jnp.float32),
        pltpu.VMEM((_V,), jnp.int32),
        pltpu.VMEM((_PPW,), jnp.float32),
        pltpu.VMEM((_LANES,), jnp.float32),
        pltpu.VMEM_SHARED((_V,), jnp.float32),
        pltpu.SemaphoreType.DMA,
        pltpu.SemaphoreType.DMA,
        pltpu.SemaphoreType.DMA,
        pltpu.SemaphoreType.DMA,
    ],
)(_body)


@jax.jit
def kernel(text, W, b):
    ids = text.astype(jnp.int32).T
    pad = jnp.zeros((_B, _SPAD - _S), jnp.int32)
    ids_flat = jnp.concatenate([ids, pad], axis=1).reshape(_B * _SPAD)
    bias16 = jnp.broadcast_to(b.astype(jnp.float32), (_LANES,))
    out = _mnb_sc(ids_flat, W.reshape(_V), bias16)
    return out.reshape(_B, 1)


# speedup vs baseline: 1.0108x; 1.0018x over previous
import functools

import jax
import jax.numpy as jnp
from jax import lax
from jax.experimental import pallas as pl
from jax.experimental.pallas import tpu as pltpu
from jax.experimental.pallas import tpu_sc as plsc

_V = 100000
_S = 200
_B = 1024
_LANES = 16
_SPAD = 208
_NCH = _SPAD // _LANES
_NVALID = _S - (_NCH - 1) * _LANES
_NW = 32
_PPW = _B // _NW
_IDS_PER_W = _PPW * _SPAD
_NGRP = 4
_PPG = _PPW // _NGRP
_IDS_PER_G = _PPG * _SPAD


def _body(ids_hbm, w_hbm, bias_hbm, out_hbm,
          ids_v, vals_v, scratch_v, outbuf_v, bias_v,
          wshared, sem0, sem1, sem2, sem3):
    sems = (sem0, sem1, sem2, sem3)
    sid = lax.axis_index("s")
    wid = sid * 2 + lax.axis_index("c")
    base = wid * _IDS_PER_W

    @pl.when(sid == 0)
    def _stage_w():
        pltpu.sync_copy(w_hbm, wshared)

    pltpu.sync_copy(ids_hbm.at[pl.ds(base, _IDS_PER_W)], ids_v)
    pltpu.sync_copy(bias_hbm, bias_v)
    plsc.subcore_barrier()
    gathers = [
        pltpu.async_copy(
            wshared.at[ids_v.at[pl.ds(g * _IDS_PER_G, _IDS_PER_G)]],
            vals_v.at[pl.ds(g * _IDS_PER_G, _IDS_PER_G)], sems[g])
        for g in range(_NGRP)
    ]
    lane = lax.iota(jnp.int32, _LANES)
    valid_last = lane < _NVALID
    bvec = bias_v[...]

    def phrase(p, carry):
        pbase = p * _SPAD
        ids = []
        for k in range(_NCH):
            ids16 = ids_v[pl.ds(pbase + k * _LANES, _LANES)]
            ids.append(ids16)
            tags16 = lane + (pbase + k * _LANES)
            if k == _NCH - 1:
                plsc.store_scatter(scratch_v, [ids16], tags16,
                                   mask=valid_last)
            else:
                plsc.store_scatter(scratch_v, [ids16], tags16)
        acc = jnp.zeros((_LANES,), jnp.float32)
        for k in range(_NCH):
            tags16 = lane + (pbase + k * _LANES)
            r16 = plsc.load_gather(scratch_v, [ids[k]])
            v16 = vals_v[pl.ds(pbase + k * _LANES, _LANES)]
            m16 = r16 == tags16
            if k == _NCH - 1:
                m16 = m16 & valid_last
            acc = acc + jnp.where(m16, v16, 0.0)
        tot = jnp.sum(acc)
        out16 = jnp.full((_LANES,), tot, jnp.float32) + bvec
        plsc.store_scatter(outbuf_v, [jnp.full((_LANES,), p, jnp.int32)],
                           out16, mask=lane == 0)
        return carry

    for g in range(_NGRP):
        gathers[g].wait()
        lax.fori_loop(g * _PPG, (g + 1) * _PPG, phrase, 0)
    pltpu.sync_copy(outbuf_v, out_hbm.at[pl.ds(wid * _PPW, _PPW)])


_mnb_sc = functools.partial(
    pl.kernel,
    out_type=jax.ShapeDtypeStruct((_B,), jnp.float32),
    mesh=plsc.VectorSubcoreMesh(core_axis_name="c", subcore_axis_name="s"),
    compiler_params=pltpu.CompilerParams(needs_layout_passes=False),
    scratch_types=[
        pltpu.VMEM((_IDS_PER_W,), jnp.int32),
        pltpu.VMEM((_IDS_PER_W,), jnp.float32),
        pltpu.VMEM((_V,), jnp.int32),
        pltpu.VMEM((_PPW,), jnp.float32),
        pltpu.VMEM((_LANES,), jnp.float32),
        pltpu.VMEM_SHARED((_V,), jnp.float32),
        pltpu.SemaphoreType.DMA,
        pltpu.SemaphoreType.DMA,
        pltpu.SemaphoreType.DMA,
        pltpu.SemaphoreType.DMA,
    ],
)(_body)


@jax.jit
def kernel(text, W, b):
    ids = text.astype(jnp.int32).T
    pad = jnp.zeros((_B, _SPAD - _S), jnp.int32)
    ids_flat = jnp.concatenate([ids, pad], axis=1).reshape(_B * _SPAD)
    bias16 = jnp.broadcast_to(b.astype(jnp.float32), (_LANES,))
    out = _mnb_sc(ids_flat, W.reshape(_V), bias16)
    return out.reshape(_B, 1)
